# baseline (speedup 1.0000x reference)
import functools

import jax
import jax.numpy as jnp
from jax.experimental import pallas as pl
from jax.experimental.pallas import tpu as pltpu

NB = 8
NK = 8

_DNT = (((0,), (0,)), ((), ()))
_DNM = (((1,), (0,)), ((), ()))


def _bf16_dot(w_ref, a_ref):
    w = w_ref[...].astype(jnp.bfloat16)
    a = a_ref[...].astype(jnp.bfloat16)
    return jax.lax.dot_general(w, a, _DNT, preferred_element_type=jnp.float32)


def _nnue_body(pov_ref, w_ref, bk_ref, waW_ref, wab_ref, baW_ref, bab_ref,
               f0W_ref, f0b_ref, f1W_ref, f1b_ref, f2W_ref, f2b_ref,
               out_ref, accw_ref, accb_ref, *, block_b):
    k = pl.program_id(0)
    b = pl.program_id(1)
    nk = pl.num_programs(0)

    pw = _bf16_dot(waW_ref, w_ref)
    pb = _bf16_dot(baW_ref, bk_ref)

    cols = pl.ds(b * block_b, block_b)

    @pl.when(k == 0)
    def _init():
        accw_ref[:, cols] = pw
        accb_ref[:, cols] = pb

    @pl.when(k > 0)
    def _accum():
        accw_ref[:, cols] += pw
        accb_ref[:, cols] += pb

    @pl.when(k == nk - 1)
    def _epilogue():
        w256 = accw_ref[:, cols] + wab_ref[...][:, None]
        b256 = accb_ref[:, cols] + bab_ref[...][:, None]
        p = pov_ref[...][None, :]
        x0 = jnp.maximum(p * w256 + (1.0 - p) * b256, 0.0)
        x1 = jnp.maximum(p * b256 + (1.0 - p) * w256, 0.0)
        f0 = f0W_ref[...]
        h = f0.shape[1] // 2
        y = (jax.lax.dot_general(f0[:, :h], x0, _DNM,
                                 preferred_element_type=jnp.float32)
             + jax.lax.dot_general(f0[:, h:], x1, _DNM,
                                   preferred_element_type=jnp.float32)
             + f0b_ref[...][:, None])
        y = jnp.maximum(y, 0.0)
        z = jax.lax.dot_general(f1W_ref[...], y, _DNM,
                                preferred_element_type=jnp.float32)
        z = jnp.maximum(z + f1b_ref[...][:, None], 0.0)
        o = jax.lax.dot_general(f2W_ref[...], z, _DNM,
                                preferred_element_type=jnp.float32)
        out_ref[cols] = o.reshape(z.shape[1]) + f2b_ref[0]


def kernel(pov, white, black, wa_W, wa_b, ba_W, ba_b,
           fc0_W, fc0_b, fc1_W, fc1_b, fc2_W, fc2_b):
    B, K = white.shape
    H = wa_W.shape[0]
    block_b = B // NB
    block_k = K // NK

    whiteT = white.T
    blackT = black.T
    waWT = wa_W.T
    baWT = ba_W.T
    pov1 = pov.reshape(B)

    grid = (NK, NB)
    full = lambda arr: pl.BlockSpec(arr.shape, lambda k, b: (0,) * arr.ndim)

    out = pl.pallas_call(
        functools.partial(_nnue_body, block_b=block_b),
        grid=grid,
        in_specs=[
            pl.BlockSpec((block_b,), lambda k, b: (b,)),
            pl.BlockSpec((block_k, block_b), lambda k, b: (k, b)),
            pl.BlockSpec((block_k, block_b), lambda k, b: (k, b)),
            pl.BlockSpec((block_k, H), lambda k, b: (k, 0)),
            full(wa_b),
            pl.BlockSpec((block_k, H), lambda k, b: (k, 0)),
            full(ba_b),
            full(fc0_W), full(fc0_b),
            full(fc1_W), full(fc1_b),
            full(fc2_W),
            pl.BlockSpec(memory_space=pltpu.SMEM),
        ],
        out_specs=pl.BlockSpec((B,), lambda k, b: (0,)),
        out_shape=jax.ShapeDtypeStruct((B,), jnp.float32),
        scratch_shapes=[
            pltpu.VMEM((H, B), jnp.float32),
            pltpu.VMEM((H, B), jnp.float32),
        ],
        compiler_params=pltpu.CompilerParams(
            dimension_semantics=("arbitrary", "arbitrary"),
        ),
    )(pov1, whiteT, blackT, waWT, wa_b, baWT, ba_b,
      fc0_W, fc0_b, fc1_W, fc1_b, fc2_W, fc2_b)
    return out.reshape(B, 1)


# speedup vs baseline: 1.3042x; 1.3042x over previous
import functools

import jax
import jax.numpy as jnp
from jax.experimental import pallas as pl
from jax.experimental.pallas import tpu as pltpu

NB = 2
NK = 8

_DNT = (((0,), (0,)), ((), ()))
_DNM = (((1,), (0,)), ((), ()))


def _bf16_dot(w_ref, a_ref):
    w = w_ref[...].astype(jnp.bfloat16)
    a = a_ref[...].astype(jnp.bfloat16)
    return jax.lax.dot_general(w, a, _DNT, preferred_element_type=jnp.float32)


def _nnue_body(pov_ref, w_ref, bk_ref, waW_ref, wab_ref, baW_ref, bab_ref,
               f0W_ref, f0b_ref, f1W_ref, f1b_ref, f2W_ref, f2b_ref,
               out_ref, accw_ref, accb_ref, *, block_b):
    k = pl.program_id(0)
    b = pl.program_id(1)
    nk = pl.num_programs(0)

    pw = _bf16_dot(waW_ref, w_ref)
    pb = _bf16_dot(baW_ref, bk_ref)

    cols = pl.ds(b * block_b, block_b)

    @pl.when(k == 0)
    def _init():
        accw_ref[:, cols] = pw
        accb_ref[:, cols] = pb

    @pl.when(k > 0)
    def _accum():
        accw_ref[:, cols] += pw
        accb_ref[:, cols] += pb

    @pl.when(k == nk - 1)
    def _epilogue():
        w256 = accw_ref[:, cols] + wab_ref[...][:, None]
        b256 = accb_ref[:, cols] + bab_ref[...][:, None]
        p = pov_ref[...][None, :]
        x0 = jnp.maximum(p * w256 + (1.0 - p) * b256, 0.0)
        x1 = jnp.maximum(p * b256 + (1.0 - p) * w256, 0.0)
        f0 = f0W_ref[...]
        h = f0.shape[1] // 2
        y = (jax.lax.dot_general(f0[:, :h], x0, _DNM,
                                 preferred_element_type=jnp.float32)
             + jax.lax.dot_general(f0[:, h:], x1, _DNM,
                                   preferred_element_type=jnp.float32)
             + f0b_ref[...][:, None])
        y = jnp.maximum(y, 0.0)
        z = jax.lax.dot_general(f1W_ref[...], y, _DNM,
                                preferred_element_type=jnp.float32)
        z = jnp.maximum(z + f1b_ref[...][:, None], 0.0)
        o = jax.lax.dot_general(f2W_ref[...], z, _DNM,
                                preferred_element_type=jnp.float32)
        out_ref[cols] = o.reshape(z.shape[1]) + f2b_ref[0]


def kernel(pov, white, black, wa_W, wa_b, ba_W, ba_b,
           fc0_W, fc0_b, fc1_W, fc1_b, fc2_W, fc2_b):
    B, K = white.shape
    H = wa_W.shape[0]
    block_b = B // NB
    block_k = K // NK

    whiteT = white.T
    blackT = black.T
    waWT = wa_W.T
    baWT = ba_W.T
    pov1 = pov.reshape(B)

    grid = (NK, NB)
    full = lambda arr: pl.BlockSpec(arr.shape, lambda k, b: (0,) * arr.ndim)

    out = pl.pallas_call(
        functools.partial(_nnue_body, block_b=block_b),
        grid=grid,
        in_specs=[
            pl.BlockSpec((block_b,), lambda k, b: (b,)),
            pl.BlockSpec((block_k, block_b), lambda k, b: (k, b)),
            pl.BlockSpec((block_k, block_b), lambda k, b: (k, b)),
            pl.BlockSpec((block_k, H), lambda k, b: (k, 0)),
            full(wa_b),
            pl.BlockSpec((block_k, H), lambda k, b: (k, 0)),
            full(ba_b),
            full(fc0_W), full(fc0_b),
            full(fc1_W), full(fc1_b),
            full(fc2_W),
            pl.BlockSpec(memory_space=pltpu.SMEM),
        ],
        out_specs=pl.BlockSpec((B,), lambda k, b: (0,)),
        out_shape=jax.ShapeDtypeStruct((B,), jnp.float32),
        scratch_shapes=[
            pltpu.VMEM((H, B), jnp.float32),
            pltpu.VMEM((H, B), jnp.float32),
        ],
        compiler_params=pltpu.CompilerParams(
            dimension_semantics=("arbitrary", "arbitrary"),
            vmem_limit_bytes=110 * 1024 * 1024,
        ),
    )(pov1, whiteT, blackT, waWT, wa_b, baWT, ba_b,
      fc0_W, fc0_b, fc1_W, fc1_b, fc2_W, fc2_b)
    return out.reshape(B, 1)
